# baseline (speedup 1.0000x reference)
import functools

import jax
import jax.numpy as jnp
from jax import lax
from jax.experimental import pallas as pl
from jax.experimental.pallas import tpu as pltpu
from jax.experimental.pallas import tpu_sc as plsc


def _sc_gather(table, idx3, n_rows):
    V, D = table.shape
    NW, n_ch, CH = idx3.shape
    per_w = n_ch * CH

    mesh = plsc.VectorSubcoreMesh(core_axis_name="c", subcore_axis_name="s")
    NBUF = 10

    @functools.partial(
        pl.kernel,
        mesh=mesh,
        compiler_params=pltpu.CompilerParams(use_tc_tiling_on_sc=False),
        out_type=jax.ShapeDtypeStruct((n_rows, D), jnp.float32),
        scratch_types=(
            [pltpu.VMEM((n_ch, CH), jnp.int32)]
            + [pltpu.VMEM((CH, D), jnp.float32) for _ in range(NBUF)]
            + [pltpu.SemaphoreType.DMA for _ in range(2 * NBUF)]
        ),
    )
    def k(table_hbm, idx_hbm, out_hbm, idx_v, *bufs_and_sems):
        rows = bufs_and_sems[:NBUF]
        gsem = bufs_and_sems[NBUF:2 * NBUF]
        wsem = bufs_and_sems[2 * NBUF:]
        wid = lax.axis_index("s") * 2 + lax.axis_index("c")
        base_row = wid * per_w

        pltpu.sync_copy(idx_hbm.at[wid], idx_v)

        for b in range(NBUF):
            pltpu.async_copy(table_hbm.at[idx_v.at[b]], rows[b], gsem[b])

        def body(i, _):
            for b in range(NBUF):
                j = i * NBUF + b
                pltpu.make_async_copy(
                    table_hbm.at[idx_v.at[0]], rows[b], gsem[b]
                ).wait()
                dst = out_hbm.at[pl.ds(base_row + j * CH, CH)]
                pltpu.async_copy(rows[b], dst, wsem[b])
                jn = j + NBUF

                @pl.when(jn < n_ch)
                def _():
                    pltpu.make_async_copy(rows[b], dst, wsem[b]).wait()
                    pltpu.async_copy(
                        table_hbm.at[idx_v.at[jn]], rows[b], gsem[b]
                    )
            return 0

        lax.fori_loop(0, n_ch // NBUF, body, 0)

        for b in range(NBUF):
            j = n_ch - NBUF + b
            pltpu.make_async_copy(
                rows[b], out_hbm.at[pl.ds(base_row + j * CH, CH)], wsem[b]
            ).wait()

    return k(table, idx3)


def _tc_loss(pairs, t_even, t_odd, n_rows):
    M, W = pairs.shape
    G, _, BLK = t_even.shape
    H = W // 2

    def body(lg_ref, te_ref, to_ref, out_ref):
        @pl.when(pl.program_id(0) == 0)
        def _():
            out_ref[...] = jnp.zeros((1, 1), jnp.float32)

        lg = lg_ref[...]
        t0 = te_ref[0, 0, :]
        t1 = to_ref[0, 0, :]
        col = lax.broadcasted_iota(jnp.int32, (BLK, W), 1)
        left = col < H
        neg = jnp.float32(-1e30)
        m_a = jnp.max(jnp.where(left, lg, neg), axis=1)
        m_b = jnp.max(jnp.where(left, neg, lg), axis=1)
        m = jnp.where(left, m_a[:, None], m_b[:, None])
        e = jnp.exp(lg - m)
        s_a = jnp.sum(jnp.where(left, e, 0.0), axis=1)
        s_b = jnp.sum(jnp.where(left, 0.0, e), axis=1)
        lse2 = jnp.log(s_a) + m_a + jnp.log(s_b) + m_b
        hit = (col == t0[:, None]) | (col == t1[:, None] + H)
        tv2 = jnp.sum(jnp.where(hit, lg, 0.0), axis=1)
        out_ref[...] += jnp.sum(lse2 - tv2).reshape(1, 1)

    loss_sum = pl.pallas_call(
        body,
        grid=(G,),
        in_specs=[
            pl.BlockSpec((BLK, W), lambda i: (i, 0)),
            pl.BlockSpec((1, 1, BLK), lambda i: (i, 0, 0)),
            pl.BlockSpec((1, 1, BLK), lambda i: (i, 0, 0)),
        ],
        out_specs=pl.BlockSpec((1, 1), lambda i: (0, 0)),
        out_shape=jax.ShapeDtypeStruct((1, 1), jnp.float32),
    )(pairs, t_even, t_odd)
    return loss_sum[0, 0] / n_rows


def kernel(inputs, targets, table):
    B, T = inputs.shape
    V, D = table.shape
    N = B * T
    CH = 128
    NW = 32
    idx3 = inputs.reshape(NW, N // (NW * CH), CH)

    logits2 = _sc_gather(table, idx3, N)

    pairs = logits2.reshape(N // 2, 2 * D)
    BLK = 640
    G = (N // 2) // BLK
    tp = targets.reshape(N // 2, 2)
    t_even = tp[:, 0].reshape(G, 1, BLK)
    t_odd = tp[:, 1].reshape(G, 1, BLK)
    loss = _tc_loss(pairs, t_even, t_odd, N)

    return logits2.reshape(B, T, D), loss


# speedup vs baseline: 1.0457x; 1.0457x over previous
import functools

import jax
import jax.numpy as jnp
from jax import lax
from jax.experimental import pallas as pl
from jax.experimental.pallas import tpu as pltpu
from jax.experimental.pallas import tpu_sc as plsc


def _sc_gather(tablef, idx3, n_rows):
    V2, D2 = tablef.shape
    NW, n_ch, CH = idx3.shape
    per_w = n_ch * CH

    mesh = plsc.VectorSubcoreMesh(core_axis_name="c", subcore_axis_name="s")
    NBUF = 5

    @functools.partial(
        pl.kernel,
        mesh=mesh,
        out_type=jax.ShapeDtypeStruct((n_rows, D2), jnp.float32),
        scratch_types=(
            [pltpu.VMEM((n_ch, CH), jnp.int32)]
            + [pltpu.VMEM((CH, D2), jnp.float32) for _ in range(NBUF)]
            + [pltpu.SemaphoreType.DMA for _ in range(2 * NBUF)]
        ),
    )
    def k(table_hbm, idx_hbm, out_hbm, idx_v, *bufs_and_sems):
        rows = bufs_and_sems[:NBUF]
        gsem = bufs_and_sems[NBUF:2 * NBUF]
        wsem = bufs_and_sems[2 * NBUF:]
        wid = lax.axis_index("s") * 2 + lax.axis_index("c")
        base_row = wid * per_w

        pltpu.sync_copy(idx_hbm.at[wid], idx_v)

        for b in range(NBUF):
            pltpu.async_copy(table_hbm.at[idx_v.at[b]], rows[b], gsem[b])

        def body(i, _):
            for b in range(NBUF):
                j = i * NBUF + b
                pltpu.make_async_copy(
                    table_hbm.at[idx_v.at[0]], rows[b], gsem[b]
                ).wait()
                dst = out_hbm.at[pl.ds(base_row + j * CH, CH)]
                pltpu.async_copy(rows[b], dst, wsem[b])
                jn = j + NBUF

                @pl.when(jn < n_ch)
                def _():
                    pltpu.make_async_copy(rows[b], dst, wsem[b]).wait()
                    pltpu.async_copy(
                        table_hbm.at[idx_v.at[jn]], rows[b], gsem[b]
                    )
            return 0

        lax.fori_loop(0, n_ch // NBUF, body, 0)

        for b in range(NBUF):
            j = n_ch - NBUF + b
            pltpu.make_async_copy(
                rows[b], out_hbm.at[pl.ds(base_row + j * CH, CH)], wsem[b]
            ).wait()

    return k(tablef, idx3)


def _tc_select_loss(g2, targets3, parity3, n_rows, d):
    N, D2 = g2.shape
    G, _, BLK = targets3.shape

    def body(g_ref, tg_ref, pr_ref, lg_ref, loss_ref):
        @pl.when(pl.program_id(0) == 0)
        def _():
            loss_ref[...] = jnp.zeros((1, 1), jnp.float32)

        g = g_ref[...]
        tg = tg_ref[0, 0, :]
        pr = pr_ref[0, 0, :]
        lg = jnp.where(pr[:, None] == 0, g[:, :d], g[:, d:])
        lg_ref[...] = lg
        m = jnp.max(lg, axis=1, keepdims=True)
        s = jnp.sum(jnp.exp(lg - m), axis=1)
        lse = jnp.log(s) + m[:, 0]
        col = lax.broadcasted_iota(jnp.int32, (BLK, d), 1)
        tv = jnp.sum(jnp.where(col == tg[:, None], lg, 0.0), axis=1)
        loss_ref[...] += jnp.sum(lse - tv).reshape(1, 1)

    logits2, loss_sum = pl.pallas_call(
        body,
        grid=(G,),
        in_specs=[
            pl.BlockSpec((BLK, D2), lambda i: (i, 0)),
            pl.BlockSpec((1, 1, BLK), lambda i: (i, 0, 0)),
            pl.BlockSpec((1, 1, BLK), lambda i: (i, 0, 0)),
        ],
        out_specs=[
            pl.BlockSpec((BLK, d), lambda i: (i, 0)),
            pl.BlockSpec((1, 1), lambda i: (0, 0)),
        ],
        out_shape=[
            jax.ShapeDtypeStruct((N, d), jnp.float32),
            jax.ShapeDtypeStruct((1, 1), jnp.float32),
        ],
    )(g2, targets3, parity3)
    return logits2, loss_sum[0, 0] / n_rows


def kernel(inputs, targets, table):
    B, T = inputs.shape
    V, D = table.shape
    N = B * T
    CH = 128
    NW = 32

    tablef = table.reshape(V // 2, 2 * D)
    flat = inputs.reshape(N)
    idx3 = (flat >> 1).reshape(NW, N // (NW * CH), CH)

    g2 = _sc_gather(tablef, idx3, N)

    BLK = 2048
    Gn = N // BLK
    targets3 = targets.reshape(Gn, 1, BLK)
    parity3 = (flat & 1).reshape(Gn, 1, BLK)
    logits2, loss = _tc_select_loss(g2, targets3, parity3, N, D)

    return logits2.reshape(B, T, D), loss
